# baseline (speedup 1.0000x reference)
import functools

import jax
import jax.numpy as jnp
from jax import lax
from jax.experimental import pallas as pl
from jax.experimental.pallas import tpu as pltpu
from jax.experimental.pallas import tpu_sc as plsc

_NC, _NS, _LANES = 2, 16, 16
_NW = _NC * _NS
_LN2 = 0.6931471805599453
_LOG2_CLAMP = -144.26950408889634

_P6 = (
    5.065332970843883e-06,
    1.4423954486846924,
    -0.716986894607544,
    0.4538562297821045,
    -0.2723531723022461,
    0.11790518462657928,
    -0.024825606495141983,
)


def _sw_log2(x):
    bits = lax.bitcast_convert_type(x, jnp.int32)
    ef = lax.convert_element_type(
        lax.shift_right_arithmetic(bits, 23) - 127, jnp.float32)
    frac = lax.bitcast_convert_type(
        lax.bitwise_or(lax.bitwise_and(bits, 0x7FFFFF), 0x3F800000),
        jnp.float32) - 1.0
    acc = jnp.float32(_P6[6])
    for c in _P6[5::-1]:
        acc = acc * frac + jnp.float32(c)
    return ef + acc


def _make_sc_kernel(B, L, chunk_rows):
    rows_per_w = B // _NW
    nchunk = rows_per_w // chunk_rows
    nfull = L // _LANES
    tail = L - nfull * _LANES

    mesh = plsc.VectorSubcoreMesh(core_axis_name="c", subcore_axis_name="s")

    @functools.partial(
        pl.kernel,
        mesh=mesh,
        out_type=[
            jax.ShapeDtypeStruct((_NW, _LANES), jnp.float32),
            jax.ShapeDtypeStruct((_NW, _LANES), jnp.float32),
        ],
        scratch_types=[
            pltpu.VMEM((chunk_rows, L), jnp.float32),
            pltpu.VMEM((chunk_rows, L), jnp.float32),
            pltpu.VMEM((chunk_rows, L), jnp.int32),
            pltpu.VMEM((_LANES,), jnp.float32),
            pltpu.VMEM((_LANES,), jnp.float32),
        ],
    )
    def sc_kernel(label_hbm, logits_hbm, mask_hbm, loss_out, cnt_out,
                  ybuf, pbuf, mbuf, lacc_v, cacc_v):
        wid = lax.axis_index("s") * _NC + lax.axis_index("c")
        row0 = wid * rows_per_w
        lane = lax.broadcasted_iota(jnp.int32, (_LANES,), 0)
        lane_f = lax.convert_element_type(lane, jnp.float32)
        tail_keep = jnp.clip(lane_f - jnp.float32(_LANES - tail - 1), 0.0, 1.0)

        def row_body(r, carry):
            acc, cnt = carry
            for j in range(nfull + 1):
                off = (L - _LANES) if j == nfull else j * _LANES
                y = ybuf[r, pl.ds(off, _LANES)]
                p = pbuf[r, pl.ds(off, _LANES)]
                m = mbuf[r, pl.ds(off, _LANES)]
                mf = lax.convert_element_type(m, jnp.float32)
                if j == nfull:
                    mf = mf * tail_keep
                lp = jnp.maximum(_sw_log2(p), _LOG2_CLAMP)
                l1p = jnp.maximum(_sw_log2(1.0 - p), _LOG2_CLAMP)
                v = l1p + y * (lp - l1p)
                acc = acc + mf * v
                cnt = cnt + mf
            return acc, cnt

        acc = jnp.zeros((_LANES,), jnp.float32)
        cnt = jnp.zeros((_LANES,), jnp.float32)
        for ch in range(nchunk):
            r0 = row0 + ch * chunk_rows
            pltpu.sync_copy(label_hbm.at[pl.ds(r0, chunk_rows), :], ybuf)
            pltpu.sync_copy(logits_hbm.at[pl.ds(r0, chunk_rows), :], pbuf)
            pltpu.sync_copy(mask_hbm.at[pl.ds(r0, chunk_rows), :], mbuf)
            acc, cnt = lax.fori_loop(0, chunk_rows, row_body, (acc, cnt))

        lacc_v[...] = acc * jnp.float32(_LN2)
        cacc_v[...] = cnt
        pltpu.sync_copy(lacc_v, loss_out.at[wid])
        pltpu.sync_copy(cacc_v, cnt_out.at[wid])

    return sc_kernel


def kernel(label, logits, mask):
    B, L = label.shape
    fn = _make_sc_kernel(B, L, chunk_rows=64)
    loss_p, cnt_p = fn(label, logits, mask.astype(jnp.int32))
    return -jnp.sum(loss_p) / jnp.sum(cnt_p)
